# baseline (speedup 1.0000x reference)
import jax
import jax.numpy as jnp
from jax.experimental import pallas as pl


def _body(x_ref, td_ref, tw_ref, out_ref):
    T = td_ref.shape[0]
    W = tw_ref.shape[0]
    N = x_ref.shape[2]
    colT = jnp.transpose(x_ref[0, 0], (1, 0))
    d_idx = jnp.clip(colT[1:2, :] * T, 0, T - 1).astype(jnp.int32)
    w_idx = jnp.clip(colT[2:3, :], 0, W - 1).astype(jnp.int32)

    iota_t = jax.lax.broadcasted_iota(jnp.int32, (T, N), 0)
    oh_d = (iota_t == d_idx).astype(jnp.float32)
    iota_w = jax.lax.broadcasted_iota(jnp.int32, (W, N), 0)
    oh_w = (iota_w == w_idx).astype(jnp.float32)

    acc = jax.lax.dot_general(td_ref[...], oh_d, (((0,), (0,)), ((), ())),
                              preferred_element_type=jnp.float32)
    acc = acc + jax.lax.dot_general(tw_ref[...], oh_w, (((0,), (0,)), ((), ())),
                                    preferred_element_type=jnp.float32)
    out_ref[0, :, :] = acc


def kernel(x, time_day, time_week):
    B, S, N, C = x.shape
    T, F = time_day.shape
    W = time_week.shape[0]

    out = pl.pallas_call(
        _body,
        grid=(B,),
        in_specs=[
            pl.BlockSpec((1, 1, N, C), lambda b: (b, S - 1, 0, 0)),
            pl.BlockSpec((T, F), lambda b: (0, 0)),
            pl.BlockSpec((W, F), lambda b: (0, 0)),
        ],
        out_specs=pl.BlockSpec((1, F, N), lambda b: (b, 0, 0)),
        out_shape=jax.ShapeDtypeStruct((B, F, N), jnp.float32),
    )(x, time_day, time_week)
    return out[..., None]


# speedup vs baseline: 4.5135x; 4.5135x over previous
import jax
import jax.numpy as jnp
from jax.experimental import pallas as pl


def _body(dw_ref, td_ref, tw_ref, out_ref):
    T = td_ref.shape[0]
    W = tw_ref.shape[0]
    N = dw_ref.shape[2]
    rows = dw_ref[0]
    d_idx = jnp.clip(rows[0:1, :] * T, 0, T - 1).astype(jnp.int32)
    w_idx = jnp.clip(rows[1:2, :], 0, W - 1).astype(jnp.int32)

    iota_t = jax.lax.broadcasted_iota(jnp.int32, (T, N), 0)
    oh_d = (iota_t == d_idx).astype(jnp.float32)
    iota_w = jax.lax.broadcasted_iota(jnp.int32, (W, N), 0)
    oh_w = (iota_w == w_idx).astype(jnp.float32)

    acc = jax.lax.dot_general(td_ref[...], oh_d, (((0,), (0,)), ((), ())),
                              preferred_element_type=jnp.float32)
    acc = acc + jax.lax.dot_general(tw_ref[...], oh_w, (((0,), (0,)), ((), ())),
                                    preferred_element_type=jnp.float32)
    out_ref[0, :, :] = acc


def kernel(x, time_day, time_week):
    B, S, N, C = x.shape
    T, F = time_day.shape
    W = time_week.shape[0]
    dw = jnp.transpose(x[:, -1, :, 1:3], (0, 2, 1))

    out = pl.pallas_call(
        _body,
        grid=(B,),
        in_specs=[
            pl.BlockSpec((1, 2, N), lambda b: (b, 0, 0)),
            pl.BlockSpec((T, F), lambda b: (0, 0)),
            pl.BlockSpec((W, F), lambda b: (0, 0)),
        ],
        out_specs=pl.BlockSpec((1, F, N), lambda b: (b, 0, 0)),
        out_shape=jax.ShapeDtypeStruct((B, F, N), jnp.float32),
    )(dw, time_day, time_week)
    return out[..., None]
